# baseline (speedup 1.0000x reference)
import numpy as np

import jax
import jax.numpy as jnp
from jax.experimental import pallas as pl
from jax.experimental.pallas import tpu as pltpu

_HALO = 64
_BL = 3

_T28 = (29, 32, 28, 28)
_T14 = (15, 16, 14, 14)
_T7 = (8, 8, 7, 7)

_CONVS = [(1, 2), (2, 4), (4, 8), (8, 8), (8, 16), (16, 32),
          (32, 16), (16, 8), (8, 8), (8, 4), (4, 2), (2, 1)]


def _rpad(n):
    return (n + 7) & ~7


def _conv_bases():
    bases, r = [], 0
    for _, co in _CONVS:
        bases.append(r)
        r += 9 * _rpad(co)
    lin_bases = []
    for rows in (32, 16, 32, 32):
        lin_bases.append(r)
        r += rows
    return bases, lin_bases, r


_CBASES, _LBASES, _BROWS = _conv_bases()
_WROWS = _BROWS + 32


def _flat(t):
    return t[0] * t[1]


_F28, _F14, _F7 = _flat(_T28), _flat(_T14), _flat(_T7)
_L28 = _BL * _F28
_L14 = _BL * _F14
_L7 = _BL * _F7
_LPAD = 2816


def _gap_mask(t, n):
    th, tw, h, w = t
    m = np.zeros((th, tw), np.float32)
    m[1:1 + h, 0:w] = 1.0
    return np.tile(m.reshape(1, -1), (1, n))


def _pool_sel(t_in, t_out):
    ti_h, ti_w, h, w = t_in
    to_h, to_w, h2, w2 = t_out
    S = np.zeros((ti_h * ti_w, to_h * to_w), np.float32)
    for y2 in range(h2):
        for x2 in range(w2):
            S[(1 + 2 * y2) * ti_w + 2 * x2, (1 + y2) * to_w + x2] = 1.0
    return S


def _upsample(t_in, t_out):
    ti_h, ti_w, h, w = t_in
    to_h, to_w, h2, w2 = t_out
    U = np.zeros((ti_h * ti_w, to_h * to_w), np.float32)
    for y2 in range(h2):
        for x2 in range(w2):
            U[(1 + y2 // 2) * ti_w + x2 // 2, (1 + y2) * to_w + x2] = 1.0
    return U


def _gpack(B):
    g = np.zeros((2, 4, _LPAD), np.float32)
    for core in range(2):
        g[core, 0, :_L28] = _gap_mask(_T28, _BL)
        g[core, 1, :_L14] = _gap_mask(_T14, _BL)
        g[core, 2, :_L7] = _gap_mask(_T7, _BL)
        for s in range(_BL):
            if core * _BL + s < B:
                g[core, 3, s * _F28:(s + 1) * _F28] = _gap_mask(_T28, 1)
    return g


def _cpack():
    c = np.zeros((629, 320), np.float32)
    c[0:240, 0:64] = _pool_sel(_T14, _T7)
    c[240:304, 0:240] = _upsample(_T7, _T14)
    g7 = _gap_mask(_T7, 1).reshape(-1)
    for s in range(_BL):
        c[304 + s * _F7:304 + (s + 1) * _F7, s] = g7 / 49.0
        c[624 + s, s * _F7:(s + 1) * _F7] = g7
    return c


def _ae_kernel(
    x_ref, g_ref, s28_ref, u14_ref, c_ref, wp_ref,
    enc_ref, loss_ref,
    bufa, bufb,
):
    H = _HALO
    BL = _BL
    N28, N14, N7 = _L28, _L14, _L7
    W28, W14, W7 = _T28[1], _T14[1], _T7[1]

    bufa[...] = jnp.zeros_like(bufa)
    bufb[...] = jnp.zeros_like(bufb)

    xv = x_ref[0][:, 0:N28]
    bufa[0:1, H:H + N28] = xv

    g28 = g_ref[0][0:1, 0:N28]
    g14 = g_ref[0][1:2, 0:N14]
    g7 = g_ref[0][2:3, 0:N7]
    rmask = g_ref[0][3:4, 0:N28]
    s14 = c_ref[0:240, 0:64]
    u7 = c_ref[240:304, 0:240]
    avg = c_ref[304:304 + N7, 0:BL]
    bc = c_ref[624:624 + BL, 0:N7]

    def conv3x3(src, dst, idx, Wp, N, act, gmask):
        cin, cout = _CONVS[idx]
        base, cp = _CBASES[idx], _rpad(cout)
        acc = jnp.zeros((cout, N), jnp.float32)
        for ky in range(3):
            for kx in range(3):
                d = (ky - 1) * Wp + (kx - 1)
                win = src[0:cin, H + d:H + d + N]
                r0 = base + (ky * 3 + kx) * cp
                wk = wp_ref[r0:r0 + cout, 0:cin]
                if cin <= 2:
                    for ci in range(cin):
                        acc = acc + wk[:, ci:ci + 1] * win[ci:ci + 1, :]
                else:
                    acc = acc + jnp.dot(wk, win,
                                        preferred_element_type=jnp.float32)
        acc = acc + wp_ref[_BROWS:_BROWS + cout, idx:idx + 1]
        if act == "relu":
            acc = jnp.maximum(acc, 0.0)
        elif act == "tanh":
            acc = jnp.tanh(acc)
        if gmask is not None:
            acc = acc * gmask
        dst[0:cout, H:H + N] = acc
        dst[0:cout, H + N:H + N + H] = jnp.zeros((cout, H), jnp.float32)

    def maxpool2x2(src, dst, sel, c, Wp, Fin, Fout, Nin):
        m = src[0:c, H:H + Nin]
        for d in (1, Wp, Wp + 1):
            m = jnp.maximum(m, src[0:c, H + d:H + d + Nin])
        for b in range(BL):
            blk = jnp.dot(m[:, b * Fin:(b + 1) * Fin], sel,
                          preferred_element_type=jnp.float32)
            dst[0:c, H + b * Fout:H + (b + 1) * Fout] = blk
        dst[0:c, H + BL * Fout:H + BL * Fout + H] = jnp.zeros((c, H),
                                                             jnp.float32)

    def up2x_relu(src, dst, up, c, Fin, Fout):
        for b in range(BL):
            blk = jnp.dot(src[0:c, H + b * Fin:H + (b + 1) * Fin], up,
                          preferred_element_type=jnp.float32)
            dst[0:c, H + b * Fout:H + (b + 1) * Fout] = jnp.maximum(blk, 0.0)
        dst[0:c, H + BL * Fout:H + BL * Fout + H] = jnp.zeros((c, H),
                                                             jnp.float32)

    def lin_w(j, rows, cols):
        return wp_ref[_LBASES[j]:_LBASES[j] + rows, 0:cols]

    def lin_b(j, rows):
        return wp_ref[_BROWS:_BROWS + rows, 12 + j:13 + j]

    conv3x3(bufa, bufb, 0, W28, N28, None, g28)
    conv3x3(bufb, bufa, 1, W28, N28, "relu", None)
    maxpool2x2(bufa, bufb, s28_ref[...], 4, W28, _F28, _F14, N28)
    conv3x3(bufb, bufa, 2, W14, N14, None, g14)
    conv3x3(bufa, bufb, 3, W14, N14, "relu", None)
    maxpool2x2(bufb, bufa, s14, 8, W14, _F14, _F7, N14)
    conv3x3(bufa, bufb, 4, W7, N7, None, g7)
    conv3x3(bufb, bufa, 5, W7, N7, "relu", None)

    pooled = jnp.dot(bufa[0:32, H:H + N7], avg,
                     preferred_element_type=jnp.float32)
    z1 = jnp.maximum(
        jnp.dot(lin_w(0, 32, 32), pooled, preferred_element_type=jnp.float32)
        + lin_b(0, 32), 0.0)
    enc = (jnp.dot(lin_w(1, 16, 32), z1, preferred_element_type=jnp.float32)
           + lin_b(1, 16))
    enc_ref[0] = enc.T

    d1 = jnp.maximum(
        jnp.dot(lin_w(2, 32, 16), enc, preferred_element_type=jnp.float32)
        + lin_b(2, 32), 0.0)
    d2 = (jnp.dot(lin_w(3, 32, 32), d1, preferred_element_type=jnp.float32)
          + lin_b(3, 32))
    d3 = jnp.maximum(
        jnp.dot(d2, bc, preferred_element_type=jnp.float32), 0.0)
    bufb[0:32, H:H + N7] = d3
    bufb[0:32, H + N7:H + N7 + H] = jnp.zeros((32, H), jnp.float32)

    conv3x3(bufb, bufa, 6, W7, N7, None, g7)
    conv3x3(bufa, bufb, 7, W7, N7, None, None)
    up2x_relu(bufb, bufa, u7, 8, _F7, _F14)
    conv3x3(bufa, bufb, 8, W14, N14, None, g14)
    conv3x3(bufb, bufa, 9, W14, N14, None, None)
    up2x_relu(bufa, bufb, u14_ref[...], 4, _F14, _F28)
    conv3x3(bufb, bufa, 10, W28, N28, None, g28)
    conv3x3(bufa, bufb, 11, W28, N28, "tanh", g28)

    diff = (bufb[0:1, H:H + N28] - xv) * rmask
    loss_ref[0] = jnp.sum(diff * diff, axis=1, keepdims=True) / float(784 * 5)


def kernel(x, m28, m14, m7, sel28, sel14, up7, up14, avg7, bc7,
           w00, w01, w02, w03, w04, w05, w06, w07, w08, w09,
           w10, w11, w12, w13, w14, w15, w16, w17, w18, w19,
           w20, w21, w22, w23, w24, w25, w26, w27, w28, w29,
           w30, w31):
    B = x.shape[0]

    xp = jnp.pad(x, ((0, 2 * _BL - B), (1, 0), (0, 4))).reshape(2, _L28)
    xp = jnp.pad(xp, ((0, 0), (0, _LPAD - _L28))).reshape(2, 1, _LPAD)

    conv_ws = [w00, w02, w04, w06, w08, w10, w20, w22, w24, w26, w28, w30]
    conv_bs = [w01, w03, w05, w07, w09, w11, w21, w23, w25, w27, w29, w31]
    lin_ws = [w12, w14, w16, w18]
    lin_bs = [w13, w15, w17, w19]

    wpack = jnp.zeros((_WROWS, 32), jnp.float32)
    for (ci, co), w, base in zip(_CONVS, conv_ws, _CBASES):
        blk = jnp.pad(w, ((0, 0), (0, _rpad(co) - co), (0, 32 - ci))
                      ).reshape(9 * _rpad(co), 32)
        wpack = wpack + jnp.pad(blk, ((base, _WROWS - base - blk.shape[0]),
                                      (0, 0)))
    for w, base in zip(lin_ws, _LBASES):
        wpack = wpack + jnp.pad(w, ((base, _WROWS - base - w.shape[0]),
                                    (0, 32 - w.shape[1])))
    for j, b in enumerate(conv_bs + lin_bs):
        wpack = wpack + jnp.pad(b, ((_BROWS, 32 - b.shape[0]), (j, 31 - j)))

    consts = [
        jnp.asarray(_gpack(B)),
        jnp.asarray(_pool_sel(_T28, _T14)),
        jnp.asarray(_upsample(_T14, _T28)),
        jnp.asarray(_cpack()),
    ]
    args = [xp] + consts + [wpack]

    buf_w = 2 * _HALO + _LPAD

    enc_t, loss = pl.pallas_call(
        _ae_kernel,
        grid=(2,),
        in_specs=[
            pl.BlockSpec((1, 1, _LPAD), lambda i: (i, 0, 0)),
            pl.BlockSpec((1, 4, _LPAD), lambda i: (i, 0, 0)),
            pl.BlockSpec((_F28, _F14), lambda i: (0, 0)),
            pl.BlockSpec((_F14, _F28), lambda i: (0, 0)),
            pl.BlockSpec((629, 320), lambda i: (0, 0)),
            pl.BlockSpec((_WROWS, 32), lambda i: (0, 0)),
        ],
        out_specs=(pl.BlockSpec((1, _BL, 16), lambda i: (i, 0, 0)),
                   pl.BlockSpec((1, 1, 1), lambda i: (i, 0, 0))),
        out_shape=(jax.ShapeDtypeStruct((2, _BL, 16), jnp.float32),
                   jax.ShapeDtypeStruct((2, 1, 1), jnp.float32)),
        scratch_shapes=[
            pltpu.VMEM((32, buf_w), jnp.float32),
            pltpu.VMEM((32, buf_w), jnp.float32),
        ],
        compiler_params=pltpu.CompilerParams(
            dimension_semantics=("parallel",),
            vmem_limit_bytes=32 * 1024 * 1024,
        ),
        cost_estimate=pl.CostEstimate(flops=16_000_000, transcendentals=25_000,
                                      bytes_accessed=3_000_000),
    )(*args)
    return enc_t.reshape(2 * _BL, 16)[0:B], jnp.sum(loss)


# speedup vs baseline: 1.3334x; 1.3334x over previous
import numpy as np

import jax
import jax.numpy as jnp
from jax.experimental import pallas as pl
from jax.experimental.pallas import tpu as pltpu

_HALO = 64

_T28 = (29, 32, 28, 28)
_T14 = (15, 16, 14, 14)
_T7 = (8, 8, 7, 7)

_CONVS = [(1, 2), (2, 4), (4, 8), (8, 8), (8, 16), (16, 32),
          (32, 16), (16, 8), (8, 8), (8, 4), (4, 2), (2, 1)]


def _rpad(n):
    return (n + 7) & ~7


def _conv_bases():
    bases, r = [], 0
    for _, co in _CONVS:
        bases.append(r)
        r += 9 * _rpad(co)
    lin_bases = []
    for rows in (32, 16, 32, 32):
        lin_bases.append(r)
        r += rows
    return bases, lin_bases, r


_CBASES, _LBASES, _BROWS = _conv_bases()
_WROWS = _BROWS + 32


def _flat(t):
    return t[0] * t[1]


_F28, _F14, _F7 = _flat(_T28), _flat(_T14), _flat(_T7)


def _gap_mask(t, n):
    th, tw, h, w = t
    m = np.zeros((th, tw), np.float32)
    m[1:1 + h, 0:w] = 1.0
    return np.tile(m.reshape(1, -1), (1, n))


def _pool_sel(t_in, t_out):
    ti_h, ti_w, h, w = t_in
    to_h, to_w, h2, w2 = t_out
    S = np.zeros((ti_h * ti_w, to_h * to_w), np.float32)
    for y2 in range(h2):
        for x2 in range(w2):
            S[(1 + 2 * y2) * ti_w + 2 * x2, (1 + y2) * to_w + x2] = 1.0
    return S


def _upsample(t_in, t_out):
    ti_h, ti_w, h, w = t_in
    to_h, to_w, h2, w2 = t_out
    U = np.zeros((ti_h * ti_w, to_h * to_w), np.float32)
    for y2 in range(h2):
        for x2 in range(w2):
            U[(1 + y2 // 2) * ti_w + x2 // 2, (1 + y2) * to_w + x2] = 1.0
    return U


def _gpack(B):
    g = np.zeros((3, B * _F28), np.float32)
    g[0] = _gap_mask(_T28, B)
    g[1, :B * _F14] = _gap_mask(_T14, B)
    g[2, :B * _F7] = _gap_mask(_T7, B)
    return g


def _cpack(B):
    c = np.zeros((629, 320), np.float32)
    c[0:240, 0:64] = _pool_sel(_T14, _T7)
    c[240:304, 0:240] = _upsample(_T7, _T14)
    g7 = _gap_mask(_T7, 1).reshape(-1)
    for s in range(B):
        c[304 + s * _F7:304 + (s + 1) * _F7, s] = g7 / 49.0
        c[624 + s, s * _F7:(s + 1) * _F7] = g7
    return c


def _ae_kernel(
    x_ref, g_ref, s28_ref, u14_ref, c_ref, wp_ref,
    enc_ref, loss_ref,
    bufa, bufb,
):
    H = _HALO
    B = enc_ref.shape[0]
    N28, N14, N7 = B * _F28, B * _F14, B * _F7
    W28, W14, W7 = _T28[1], _T14[1], _T7[1]

    bufa[...] = jnp.zeros_like(bufa)
    bufb[...] = jnp.zeros_like(bufb)

    xv = x_ref[...]
    bufa[0:1, H:H + N28] = xv

    g28 = g_ref[0:1, 0:N28]
    g14 = g_ref[1:2, 0:N14]
    g7 = g_ref[2:3, 0:N7]
    s14 = c_ref[0:240, 0:64]
    u7 = c_ref[240:304, 0:240]
    avg = c_ref[304:304 + N7, 0:B]
    bc = c_ref[624:624 + B, 0:N7]

    def conv3x3(src, dst, idx, Wp, N, act, gmask):
        cin, cout = _CONVS[idx]
        base, cp = _CBASES[idx], _rpad(cout)
        acc = jnp.zeros((cout, N), jnp.float32)
        for ky in range(3):
            for kx in range(3):
                d = (ky - 1) * Wp + (kx - 1)
                win = src[0:cin, H + d:H + d + N]
                r0 = base + (ky * 3 + kx) * cp
                wk = wp_ref[r0:r0 + cout, 0:cin]
                if cin <= 2:
                    for ci in range(cin):
                        acc = acc + wk[:, ci:ci + 1] * win[ci:ci + 1, :]
                else:
                    acc = acc + jnp.dot(wk, win,
                                        preferred_element_type=jnp.float32)
        acc = acc + wp_ref[_BROWS:_BROWS + cout, idx:idx + 1]
        if act == "relu":
            acc = jnp.maximum(acc, 0.0)
        elif act == "tanh":
            acc = jnp.tanh(acc)
        if gmask is not None:
            acc = acc * gmask
        dst[0:cout, H:H + N] = acc
        dst[0:cout, H + N:H + N + H] = jnp.zeros((cout, H), jnp.float32)

    def maxpool2x2(src, dst, sel, c, Wp, Fin, Fout, Nin):
        m = src[0:c, H:H + Nin]
        for d in (1, Wp, Wp + 1):
            m = jnp.maximum(m, src[0:c, H + d:H + d + Nin])
        for b in range(B):
            blk = jnp.dot(m[:, b * Fin:(b + 1) * Fin], sel,
                          preferred_element_type=jnp.float32)
            dst[0:c, H + b * Fout:H + (b + 1) * Fout] = blk
        dst[0:c, H + B * Fout:H + B * Fout + H] = jnp.zeros((c, H),
                                                           jnp.float32)

    def up2x_relu(src, dst, up, c, Fin, Fout):
        for b in range(B):
            blk = jnp.dot(src[0:c, H + b * Fin:H + (b + 1) * Fin], up,
                          preferred_element_type=jnp.float32)
            dst[0:c, H + b * Fout:H + (b + 1) * Fout] = jnp.maximum(blk, 0.0)
        dst[0:c, H + B * Fout:H + B * Fout + H] = jnp.zeros((c, H),
                                                           jnp.float32)

    def lin_w(j, rows, cols):
        return wp_ref[_LBASES[j]:_LBASES[j] + rows, 0:cols]

    def lin_b(j, rows):
        return wp_ref[_BROWS:_BROWS + rows, 12 + j:13 + j]

    conv3x3(bufa, bufb, 0, W28, N28, None, g28)
    conv3x3(bufb, bufa, 1, W28, N28, "relu", None)
    maxpool2x2(bufa, bufb, s28_ref[...], 4, W28, _F28, _F14, N28)
    conv3x3(bufb, bufa, 2, W14, N14, None, g14)
    conv3x3(bufa, bufb, 3, W14, N14, "relu", None)
    maxpool2x2(bufb, bufa, s14, 8, W14, _F14, _F7, N14)
    conv3x3(bufa, bufb, 4, W7, N7, None, g7)
    conv3x3(bufb, bufa, 5, W7, N7, "relu", None)

    pooled = jnp.dot(bufa[0:32, H:H + N7], avg,
                     preferred_element_type=jnp.float32)
    z1 = jnp.maximum(
        jnp.dot(lin_w(0, 32, 32), pooled, preferred_element_type=jnp.float32)
        + lin_b(0, 32), 0.0)
    enc = (jnp.dot(lin_w(1, 16, 32), z1, preferred_element_type=jnp.float32)
           + lin_b(1, 16))
    enc_ref[...] = enc.T

    d1 = jnp.maximum(
        jnp.dot(lin_w(2, 32, 16), enc, preferred_element_type=jnp.float32)
        + lin_b(2, 32), 0.0)
    d2 = (jnp.dot(lin_w(3, 32, 32), d1, preferred_element_type=jnp.float32)
          + lin_b(3, 32))
    d3 = jnp.maximum(
        jnp.dot(d2, bc, preferred_element_type=jnp.float32), 0.0)
    bufb[0:32, H:H + N7] = d3
    bufb[0:32, H + N7:H + N7 + H] = jnp.zeros((32, H), jnp.float32)

    conv3x3(bufb, bufa, 6, W7, N7, None, g7)
    conv3x3(bufa, bufb, 7, W7, N7, None, None)
    up2x_relu(bufb, bufa, u7, 8, _F7, _F14)
    conv3x3(bufa, bufb, 8, W14, N14, None, g14)
    conv3x3(bufb, bufa, 9, W14, N14, None, None)
    up2x_relu(bufa, bufb, u14_ref[...], 4, _F14, _F28)
    conv3x3(bufb, bufa, 10, W28, N28, None, g28)
    conv3x3(bufa, bufb, 11, W28, N28, "tanh", g28)

    decoded = bufb[0:1, H:H + N28]
    diff = decoded - xv
    loss_ref[...] = jnp.sum(diff * diff, axis=1, keepdims=True) / float(784 * B)


def _zero_map(nd):
    return lambda i: (0,) * nd


def kernel(x, m28, m14, m7, sel28, sel14, up7, up14, avg7, bc7,
           w00, w01, w02, w03, w04, w05, w06, w07, w08, w09,
           w10, w11, w12, w13, w14, w15, w16, w17, w18, w19,
           w20, w21, w22, w23, w24, w25, w26, w27, w28, w29,
           w30, w31):
    B = x.shape[0]
    N28 = B * _F28

    xp = jnp.pad(x, ((0, 0), (1, 0), (0, 4))).reshape(1, N28)

    conv_ws = [w00, w02, w04, w06, w08, w10, w20, w22, w24, w26, w28, w30]
    conv_bs = [w01, w03, w05, w07, w09, w11, w21, w23, w25, w27, w29, w31]
    lin_ws = [w12, w14, w16, w18]
    lin_bs = [w13, w15, w17, w19]

    wpack = jnp.zeros((_WROWS, 32), jnp.float32)
    for (ci, co), w, base in zip(_CONVS, conv_ws, _CBASES):
        blk = jnp.pad(w, ((0, 0), (0, _rpad(co) - co), (0, 32 - ci))
                      ).reshape(9 * _rpad(co), 32)
        wpack = wpack + jnp.pad(blk, ((base, _WROWS - base - blk.shape[0]),
                                      (0, 0)))
    for w, base in zip(lin_ws, _LBASES):
        wpack = wpack + jnp.pad(w, ((base, _WROWS - base - w.shape[0]),
                                    (0, 32 - w.shape[1])))
    for j, b in enumerate(conv_bs + lin_bs):
        wpack = wpack + jnp.pad(b, ((_BROWS, 32 - b.shape[0]), (j, 31 - j)))

    consts = [
        jnp.asarray(_gpack(B)),
        jnp.asarray(_pool_sel(_T28, _T14)),
        jnp.asarray(_upsample(_T14, _T28)),
        jnp.asarray(_cpack(B)),
    ]
    args = [xp] + consts + [wpack]

    buf_w = 2 * _HALO + N28

    enc_t, loss = pl.pallas_call(
        _ae_kernel,
        grid=(1,),
        in_specs=[pl.BlockSpec(a.shape, _zero_map(a.ndim)) for a in args],
        out_specs=(pl.BlockSpec((B, 16), lambda i: (0, 0)),
                   pl.BlockSpec((1, 1), lambda i: (0, 0))),
        out_shape=(jax.ShapeDtypeStruct((B, 16), jnp.float32),
                   jax.ShapeDtypeStruct((1, 1), jnp.float32)),
        scratch_shapes=[
            pltpu.VMEM((32, buf_w), jnp.float32),
            pltpu.VMEM((32, buf_w), jnp.float32),
        ],
        compiler_params=pltpu.CompilerParams(
            dimension_semantics=("arbitrary",),
            vmem_limit_bytes=32 * 1024 * 1024,
        ),
        cost_estimate=pl.CostEstimate(flops=16_000_000, transcendentals=25_000,
                                      bytes_accessed=3_000_000),
    )(*args)
    return enc_t, loss[0, 0]


# speedup vs baseline: 1.7051x; 1.2788x over previous
import numpy as np

import jax
import jax.numpy as jnp
from jax.experimental import pallas as pl
from jax.experimental.pallas import tpu as pltpu

_HALO = 64

_T28 = (29, 32, 28, 28)
_T14 = (15, 16, 14, 14)
_T7 = (8, 8, 7, 7)

_CONVS = [(1, 2), (2, 4), (4, 8), (8, 8), (8, 16), (16, 32),
          (32, 16), (16, 8), (8, 8), (8, 4), (4, 2), (2, 1)]


def _rpad(n):
    return (n + 7) & ~7


def _conv_bases():
    bases, r = [], 0
    for _, co in _CONVS:
        bases.append(r)
        r += 9 * _rpad(co)
    lin_bases = []
    for rows in (32, 16, 32, 32):
        lin_bases.append(r)
        r += rows
    return bases, lin_bases, r


_CBASES, _LBASES, _BROWS = _conv_bases()
_WROWS = _BROWS + 32


def _flat(t):
    return t[0] * t[1]


_F28, _F14, _F7 = _flat(_T28), _flat(_T14), _flat(_T7)


def _gap_mask(t, n):
    th, tw, h, w = t
    m = np.zeros((th, tw), np.float32)
    m[1:1 + h, 0:w] = 1.0
    return np.tile(m.reshape(1, -1), (1, n))


def _pool_sel(t_in, t_out):
    ti_h, ti_w, h, w = t_in
    to_h, to_w, h2, w2 = t_out
    S = np.zeros((ti_h * ti_w, to_h * to_w), np.float32)
    for y2 in range(h2):
        for x2 in range(w2):
            S[(1 + 2 * y2) * ti_w + 2 * x2, (1 + y2) * to_w + x2] = 1.0
    return S


def _upsample(t_in, t_out):
    ti_h, ti_w, h, w = t_in
    to_h, to_w, h2, w2 = t_out
    U = np.zeros((ti_h * ti_w, to_h * to_w), np.float32)
    for y2 in range(h2):
        for x2 in range(w2):
            U[(1 + y2 // 2) * ti_w + x2 // 2, (1 + y2) * to_w + x2] = 1.0
    return U


def _gpack(B):
    g = np.zeros((3, B * _F28), np.float32)
    g[0] = _gap_mask(_T28, B)
    g[1, :B * _F14] = _gap_mask(_T14, B)
    g[2, :B * _F7] = _gap_mask(_T7, B)
    return g


def _cpack(B):
    c = np.zeros((629, 320), np.float32)
    c[0:240, 0:64] = _pool_sel(_T14, _T7)
    c[240:304, 0:240] = _upsample(_T7, _T14)
    g7 = _gap_mask(_T7, 1).reshape(-1)
    for s in range(B):
        c[304 + s * _F7:304 + (s + 1) * _F7, s] = g7 / 49.0
        c[624 + s, s * _F7:(s + 1) * _F7] = g7
    return c


def _ae_kernel(
    x_ref, g_ref, s28_ref, u14_ref, c_ref, wp_ref, bp_ref,
    enc_ref, loss_ref,
    bufa, bufb,
):
    H = _HALO
    B = enc_ref.shape[0]
    N28, N14, N7 = B * _F28, B * _F14, B * _F7
    W28, W14, W7 = _T28[1], _T14[1], _T7[1]

    bufa[...] = jnp.zeros_like(bufa)
    bufb[...] = jnp.zeros_like(bufb)

    xv = x_ref[...]
    bufa[0:1, H:H + N28] = xv

    g28 = g_ref[0:1, 0:N28]
    g14 = g_ref[1:2, 0:N14]
    g7 = g_ref[2:3, 0:N7]
    s14 = c_ref[0:240, 0:64]
    u7 = c_ref[240:304, 0:240]
    avg = c_ref[304:304 + N7, 0:B]
    bc = c_ref[624:624 + B, 0:N7]

    def conv3x3(src, dst, idx, Wp, N, act, gmask):
        cin, cout = _CONVS[idx]
        base, cp = _CBASES[idx], _rpad(cout)
        acc = jnp.zeros((cout, N), jnp.float32)
        for ky in range(3):
            for kx in range(3):
                d = (ky - 1) * Wp + (kx - 1)
                win = src[0:cin, H + d:H + d + N]
                r0 = base + (ky * 3 + kx) * cp
                wk = wp_ref[r0:r0 + cout, 0:cin]
                if cin <= 2:
                    for ci in range(cin):
                        acc = acc + wk[:, ci:ci + 1] * win[ci:ci + 1, :]
                else:
                    acc = acc + jnp.dot(wk, win,
                                        preferred_element_type=jnp.float32)
        acc = acc + bp_ref[0:cout, idx:idx + 1]
        if act == "relu":
            acc = jnp.maximum(acc, 0.0)
        elif act == "tanh":
            acc = jnp.tanh(acc)
        if gmask is not None:
            acc = acc * gmask
        dst[0:cout, H:H + N] = acc
        dst[0:cout, H + N:H + N + H] = jnp.zeros((cout, H), jnp.float32)

    def maxpool2x2(src, dst, sel, c, Wp, Fin, Fout, Nin):
        m = src[0:c, H:H + Nin]
        for d in (1, Wp, Wp + 1):
            m = jnp.maximum(m, src[0:c, H + d:H + d + Nin])
        for b in range(B):
            blk = jnp.dot(m[:, b * Fin:(b + 1) * Fin], sel,
                          preferred_element_type=jnp.float32)
            dst[0:c, H + b * Fout:H + (b + 1) * Fout] = blk
        dst[0:c, H + B * Fout:H + B * Fout + H] = jnp.zeros((c, H),
                                                           jnp.float32)

    def up2x_relu(src, dst, up, c, Fin, Fout):
        for b in range(B):
            blk = jnp.dot(src[0:c, H + b * Fin:H + (b + 1) * Fin], up,
                          preferred_element_type=jnp.float32)
            dst[0:c, H + b * Fout:H + (b + 1) * Fout] = jnp.maximum(blk, 0.0)
        dst[0:c, H + B * Fout:H + B * Fout + H] = jnp.zeros((c, H),
                                                           jnp.float32)

    def lin_w(j, rows, cols):
        return wp_ref[_LBASES[j]:_LBASES[j] + rows, 0:cols]

    def lin_b(j, rows):
        return bp_ref[0:rows, 12 + j:13 + j]

    conv3x3(bufa, bufb, 0, W28, N28, None, g28)
    conv3x3(bufb, bufa, 1, W28, N28, "relu", None)
    maxpool2x2(bufa, bufb, s28_ref[...], 4, W28, _F28, _F14, N28)
    conv3x3(bufb, bufa, 2, W14, N14, None, g14)
    conv3x3(bufa, bufb, 3, W14, N14, "relu", None)
    maxpool2x2(bufb, bufa, s14, 8, W14, _F14, _F7, N14)
    conv3x3(bufa, bufb, 4, W7, N7, None, g7)
    conv3x3(bufb, bufa, 5, W7, N7, "relu", None)

    pooled = jnp.dot(bufa[0:32, H:H + N7], avg,
                     preferred_element_type=jnp.float32)
    z1 = jnp.maximum(
        jnp.dot(lin_w(0, 32, 32), pooled, preferred_element_type=jnp.float32)
        + lin_b(0, 32), 0.0)
    enc = (jnp.dot(lin_w(1, 16, 32), z1, preferred_element_type=jnp.float32)
           + lin_b(1, 16))
    enc_ref[...] = enc.T

    d1 = jnp.maximum(
        jnp.dot(lin_w(2, 32, 16), enc, preferred_element_type=jnp.float32)
        + lin_b(2, 32), 0.0)
    d2 = (jnp.dot(lin_w(3, 32, 32), d1, preferred_element_type=jnp.float32)
          + lin_b(3, 32))
    d3 = jnp.maximum(
        jnp.dot(d2, bc, preferred_element_type=jnp.float32), 0.0)
    bufb[0:32, H:H + N7] = d3
    bufb[0:32, H + N7:H + N7 + H] = jnp.zeros((32, H), jnp.float32)

    conv3x3(bufb, bufa, 6, W7, N7, None, g7)
    conv3x3(bufa, bufb, 7, W7, N7, None, None)
    up2x_relu(bufb, bufa, u7, 8, _F7, _F14)
    conv3x3(bufa, bufb, 8, W14, N14, None, g14)
    conv3x3(bufb, bufa, 9, W14, N14, None, None)
    up2x_relu(bufa, bufb, u14_ref[...], 4, _F14, _F28)
    conv3x3(bufb, bufa, 10, W28, N28, None, g28)
    conv3x3(bufa, bufb, 11, W28, N28, "tanh", g28)

    decoded = bufb[0:1, H:H + N28]
    diff = decoded - xv
    loss_ref[...] = jnp.sum(diff * diff, axis=1, keepdims=True) / float(784 * B)


def _zero_map(nd):
    return lambda i: (0,) * nd


def kernel(x, m28, m14, m7, sel28, sel14, up7, up14, avg7, bc7,
           w00, w01, w02, w03, w04, w05, w06, w07, w08, w09,
           w10, w11, w12, w13, w14, w15, w16, w17, w18, w19,
           w20, w21, w22, w23, w24, w25, w26, w27, w28, w29,
           w30, w31):
    B = x.shape[0]
    N28 = B * _F28

    xp = jnp.pad(x, ((0, 0), (1, 0), (0, 4))).reshape(1, N28)

    conv_ws = [w00, w02, w04, w06, w08, w10, w20, w22, w24, w26, w28, w30]
    conv_bs = [w01, w03, w05, w07, w09, w11, w21, w23, w25, w27, w29, w31]
    lin_ws = [w12, w14, w16, w18]
    lin_bs = [w13, w15, w17, w19]

    wpack = jnp.zeros((_BROWS, 32), jnp.float32)
    for (ci, co), w, base in zip(_CONVS, conv_ws, _CBASES):
        blk = jnp.pad(w, ((0, 0), (0, _rpad(co) - co), (0, 32 - ci))
                      ).reshape(9 * _rpad(co), 32)
        wpack = wpack + jnp.pad(blk, ((base, _BROWS - base - blk.shape[0]),
                                      (0, 0)))
    for w, base in zip(lin_ws, _LBASES):
        wpack = wpack + jnp.pad(w, ((base, _BROWS - base - w.shape[0]),
                                    (0, 32 - w.shape[1])))
    bpack = jnp.zeros((32, 16), jnp.float32)
    for j, b in enumerate(conv_bs + lin_bs):
        bpack = bpack + jnp.pad(b, ((0, 32 - b.shape[0]), (j, 15 - j)))

    consts = [
        jnp.asarray(_gpack(B)),
        jnp.asarray(_pool_sel(_T28, _T14)),
        jnp.asarray(_upsample(_T14, _T28)),
        jnp.asarray(_cpack(B)),
    ]
    args = [xp] + consts + [wpack, bpack]

    buf_w = 2 * _HALO + N28

    enc_t, loss = pl.pallas_call(
        _ae_kernel,
        grid=(1,),
        in_specs=[pl.BlockSpec(a.shape, _zero_map(a.ndim)) for a in args],
        out_specs=(pl.BlockSpec((B, 16), lambda i: (0, 0)),
                   pl.BlockSpec((1, 1), lambda i: (0, 0))),
        out_shape=(jax.ShapeDtypeStruct((B, 16), jnp.float32),
                   jax.ShapeDtypeStruct((1, 1), jnp.float32)),
        scratch_shapes=[
            pltpu.VMEM((32, buf_w), jnp.float32),
            pltpu.VMEM((32, buf_w), jnp.float32),
        ],
        compiler_params=pltpu.CompilerParams(
            dimension_semantics=("arbitrary",),
            vmem_limit_bytes=32 * 1024 * 1024,
        ),
        cost_estimate=pl.CostEstimate(flops=16_000_000, transcendentals=25_000,
                                      bytes_accessed=3_000_000),
    )(*args)
    return enc_t, loss[0, 0]
